# baseline (speedup 1.0000x reference)
import jax
import jax.numpy as jnp
from jax.experimental import pallas as pl


def _dot(a, b):
    return jax.lax.dot_general(a, b, (((1,), (0,)), ((), ())),
                               preferred_element_type=jnp.float32)


def _geoconv_kernel(c_ref, s_ref, w1p_ref, w1s_ref, wn1_ref, w2p_ref, b2p_ref,
                    w2s_ref, wn2_ref, b2_ref, wpr_ref, bpr_ref, wc_ref,
                    bc_ref, out_ref):
    ct = jnp.transpose(c_ref[0], (1, 0))
    s = s_ref[0].astype(jnp.float32)
    X = jnp.concatenate([ct, s, jnp.ones_like(s)], axis=0)
    L = X.shape[1]

    p1 = jax.nn.relu(_dot(w1p_ref[...], X))
    n1 = jnp.concatenate(
        [jnp.zeros((p1.shape[0], 1), jnp.float32), p1[:, :-1]], axis=1)
    h1 = _dot(w1s_ref[...], X) + _dot(wn1_ref[...], n1)

    p2 = jax.nn.relu(_dot(w2p_ref[...], h1) + b2p_ref[...])
    n2 = jnp.concatenate(
        [jnp.zeros((p2.shape[0], 1), jnp.float32), p2[:, :-1]], axis=1)
    h2 = _dot(w2s_ref[...], h1) + _dot(wn2_ref[...], n2) + b2_ref[...]

    ci = jnp.tanh(_dot(wpr_ref[...], h2) + bpr_ref[...])

    y = _dot(wc_ref[0], ci)
    y += _dot(wc_ref[1], jnp.concatenate(
        [ci[:, 1:], jnp.zeros((ci.shape[0], 1), jnp.float32)], axis=1))
    y += _dot(wc_ref[2], jnp.concatenate(
        [ci[:, 2:], jnp.zeros((ci.shape[0], 2), jnp.float32)], axis=1))
    y += bc_ref[...]

    y = jnp.where(y > 0, y, jnp.exp(jnp.minimum(y, 0.0)) - 1.0)
    out_ref[0] = y[:, : L - 2]


def _fold_emb(W, b, emb):
    de = emb[1] - emb[0]
    v = de[0] * W[8] + de[1] * W[9]
    c = emb[0, 0] * W[8] + emb[0, 1] * W[9] + b
    return jnp.concatenate([W[:8], v[None, :], c[None, :]], axis=0)


def kernel(coords, current_state, emb, Wpool1, bpool1, Wself1, Wneigh1, b1,
           Wpool2, bpool2, Wself2, Wneigh2, b2, Wproc, bproc, Wconv, bconv):
    B, L, _ = coords.shape
    C = Wconv.shape[0]

    s3 = current_state.astype(jnp.int32).reshape(B, 1, L)

    w1p = _fold_emb(Wpool1, bpool1, emb).T
    w1s = _fold_emb(Wself1, b1, emb).T
    wn1 = Wneigh1.T
    w2p = Wpool2.T
    w2s = Wself2.T
    wn2 = Wneigh2.T
    wpr = Wproc.T
    wc = jnp.transpose(Wconv, (2, 0, 1))

    full = lambda shape: pl.BlockSpec(shape, lambda b: (0,) * len(shape))
    grid_spec = pl.GridSpec(
        grid=(B,),
        in_specs=[
            pl.BlockSpec((1, L, 8), lambda b: (b, 0, 0)),
            pl.BlockSpec((1, 1, L), lambda b: (b, 0, 0)),
            full(w1p.shape), full(w1s.shape), full(wn1.shape),
            full(w2p.shape), full((w2p.shape[0], 1)),
            full(w2s.shape), full(wn2.shape), full((w2s.shape[0], 1)),
            full(wpr.shape), full((wpr.shape[0], 1)),
            full(wc.shape), full((C, 1)),
        ],
        out_specs=pl.BlockSpec((1, C, L - 2), lambda b: (b, 0, 0)),
    )
    out = pl.pallas_call(
        _geoconv_kernel,
        grid_spec=grid_spec,
        out_shape=jax.ShapeDtypeStruct((B, C, L - 2), jnp.float32),
    )(coords, s3, w1p, w1s, wn1, w2p, bpool2[:, None], w2s, wn2, b2[:, None],
      wpr, bproc[:, None], wc, bconv[:, None])
    return out


# speedup vs baseline: 1.3219x; 1.3219x over previous
import jax
import jax.numpy as jnp
from jax.experimental import pallas as pl


def _dot(a, b):
    return jax.lax.dot_general(a, b, (((1,), (0,)), ((), ())),
                               preferred_element_type=jnp.float32)


def _geoconv_kernel(c_ref, s_ref, w1p_ref, w1s_ref, wn1_ref, w2p_ref, b2p_ref,
                    w2s_ref, wn2_ref, b2_ref, wpr_ref, bpr_ref, wc_ref,
                    bc_ref, out_ref):
    ct = c_ref[0]
    s = s_ref[0].astype(jnp.float32)
    X = jnp.concatenate([ct, s, jnp.ones_like(s)], axis=0)
    L = X.shape[1]

    p1 = jax.nn.relu(_dot(w1p_ref[...], X))
    n1 = jnp.concatenate(
        [jnp.zeros((p1.shape[0], 1), jnp.float32), p1[:, :-1]], axis=1)
    h1 = _dot(w1s_ref[...], X) + _dot(wn1_ref[...], n1)

    p2 = jax.nn.relu(_dot(w2p_ref[...], h1) + b2p_ref[...])
    n2 = jnp.concatenate(
        [jnp.zeros((p2.shape[0], 1), jnp.float32), p2[:, :-1]], axis=1)
    h2 = _dot(w2s_ref[...], h1) + _dot(wn2_ref[...], n2) + b2_ref[...]

    ci = jnp.tanh(_dot(wpr_ref[...], h2) + bpr_ref[...])

    y = _dot(wc_ref[0], ci)
    y += _dot(wc_ref[1], jnp.concatenate(
        [ci[:, 1:], jnp.zeros((ci.shape[0], 1), jnp.float32)], axis=1))
    y += _dot(wc_ref[2], jnp.concatenate(
        [ci[:, 2:], jnp.zeros((ci.shape[0], 2), jnp.float32)], axis=1))
    y += bc_ref[...]

    y = jnp.where(y > 0, y, jnp.exp(jnp.minimum(y, 0.0)) - 1.0)
    out_ref[0] = y[:, : L - 2]


def _fold_emb(W, b, emb):
    de = emb[1] - emb[0]
    v = de[0] * W[8] + de[1] * W[9]
    c = emb[0, 0] * W[8] + emb[0, 1] * W[9] + b
    return jnp.concatenate([W[:8], v[None, :], c[None, :]], axis=0)


def kernel(coords, current_state, emb, Wpool1, bpool1, Wself1, Wneigh1, b1,
           Wpool2, bpool2, Wself2, Wneigh2, b2, Wproc, bproc, Wconv, bconv):
    B, L, _ = coords.shape
    C = Wconv.shape[0]

    s3 = current_state.astype(jnp.int32).reshape(B, 1, L)

    w1p = _fold_emb(Wpool1, bpool1, emb).T
    w1s = _fold_emb(Wself1, b1, emb).T
    wn1 = Wneigh1.T
    w2p = Wpool2.T
    w2s = Wself2.T
    wn2 = Wneigh2.T
    wpr = Wproc.T
    wc = jnp.transpose(Wconv, (2, 0, 1))

    full = lambda shape: pl.BlockSpec(shape, lambda b: (0,) * len(shape))
    grid_spec = pl.GridSpec(
        grid=(B,),
        in_specs=[
            pl.BlockSpec((1, 8, L), lambda b: (b, 0, 0)),
            pl.BlockSpec((1, 1, L), lambda b: (b, 0, 0)),
            full(w1p.shape), full(w1s.shape), full(wn1.shape),
            full(w2p.shape), full((w2p.shape[0], 1)),
            full(w2s.shape), full(wn2.shape), full((w2s.shape[0], 1)),
            full(wpr.shape), full((wpr.shape[0], 1)),
            full(wc.shape), full((C, 1)),
        ],
        out_specs=pl.BlockSpec((1, C, L - 2), lambda b: (b, 0, 0)),
    )
    out = pl.pallas_call(
        _geoconv_kernel,
        grid_spec=grid_spec,
        out_shape=jax.ShapeDtypeStruct((B, C, L - 2), jnp.float32),
    )(jnp.transpose(coords, (0, 2, 1)), s3, w1p, w1s, wn1, w2p, bpool2[:, None], w2s, wn2, b2[:, None],
      wpr, bproc[:, None], wc, bconv[:, None])
    return out
